# baseline (speedup 1.0000x reference)
import functools

import jax
import jax.numpy as jnp
from jax import lax
from jax.experimental import pallas as pl
from jax.experimental.pallas import tpu as pltpu
from jax.experimental.pallas import tpu_sc as plsc


def _gather_kernel(B, D, b_per_w, num_cores):
    mesh = plsc.VectorSubcoreMesh(core_axis_name="c", subcore_axis_name="s")

    @functools.partial(
        pl.kernel,
        mesh=mesh,
        out_type=(
            jax.ShapeDtypeStruct((B, D), jnp.float32),
            jax.ShapeDtypeStruct((B, D), jnp.float32),
        ),
        scratch_types=[
            pltpu.VMEM((b_per_w,), jnp.int32),
            pltpu.VMEM((b_per_w,), jnp.int32),
            pltpu.VMEM((b_per_w, D), jnp.float32),
            pltpu.SemaphoreType.DMA,
        ],
    )
    def k(x_hbm, y_hbm, emb_hbm, zx_hbm, zy_hbm, xidx_v, yidx_v, rows_v, sem):
        wid = lax.axis_index("s") * num_cores + lax.axis_index("c")
        base = wid * b_per_w
        pltpu.sync_copy(x_hbm.at[pl.ds(base, b_per_w)], xidx_v)
        pltpu.sync_copy(y_hbm.at[pl.ds(base, b_per_w)], yidx_v)
        pltpu.async_copy(emb_hbm.at[xidx_v], rows_v, sem).wait()
        pltpu.sync_copy(rows_v, zx_hbm.at[pl.ds(base, b_per_w)])
        pltpu.async_copy(emb_hbm.at[yidx_v], rows_v, sem).wait()
        pltpu.sync_copy(rows_v, zy_hbm.at[pl.ds(base, b_per_w)])

    return k


def _copy_body(a_ref, o_ref):
    o_ref[...] = a_ref[...]


def _tc_copy(a):
    return pl.pallas_call(
        _copy_body,
        out_shape=jax.ShapeDtypeStruct(a.shape, a.dtype),
    )(a)


def kernel(x, y, action, emb):
    (B,) = x.shape
    V, D = emb.shape
    info = plsc.get_sparse_core_info()
    nw = info.num_cores * info.num_subcores
    b_per_w = B // nw
    k = _gather_kernel(B, D, b_per_w, info.num_cores)
    zx, zy = k(x.astype(jnp.int32), y.astype(jnp.int32), emb)
    return (zx, zy, _tc_copy(action))


# speedup vs baseline: 1.2770x; 1.2770x over previous
import functools

import jax
import jax.numpy as jnp
from jax import lax
from jax.experimental import pallas as pl
from jax.experimental.pallas import tpu as pltpu
from jax.experimental.pallas import tpu_sc as plsc


def _gather_kernel(B, D, b_per_w, num_cores):
    mesh = plsc.VectorSubcoreMesh(core_axis_name="c", subcore_axis_name="s")

    @functools.partial(
        pl.kernel,
        mesh=mesh,
        out_type=(
            jax.ShapeDtypeStruct((B, D), jnp.float32),
            jax.ShapeDtypeStruct((B, D), jnp.float32),
        ),
        scratch_types=[
            pltpu.VMEM((b_per_w,), jnp.int32),
            pltpu.VMEM((b_per_w,), jnp.int32),
            pltpu.VMEM((b_per_w, D), jnp.float32),
            pltpu.SemaphoreType.DMA,
        ],
    )
    def k(x_hbm, y_hbm, emb_hbm, zx_hbm, zy_hbm, xidx_v, yidx_v, rows_v, sem):
        wid = lax.axis_index("s") * num_cores + lax.axis_index("c")
        base = wid * b_per_w
        pltpu.sync_copy(x_hbm.at[pl.ds(base, b_per_w)], xidx_v)
        pltpu.sync_copy(y_hbm.at[pl.ds(base, b_per_w)], yidx_v)
        pltpu.async_copy(emb_hbm.at[xidx_v], rows_v, sem).wait()
        pltpu.sync_copy(rows_v, zx_hbm.at[pl.ds(base, b_per_w)])
        pltpu.async_copy(emb_hbm.at[yidx_v], rows_v, sem).wait()
        pltpu.sync_copy(rows_v, zy_hbm.at[pl.ds(base, b_per_w)])

    return k


def kernel(x, y, action, emb):
    (B,) = x.shape
    V, D = emb.shape
    info = plsc.get_sparse_core_info()
    nw = info.num_cores * info.num_subcores
    b_per_w = B // nw
    k = _gather_kernel(B, D, b_per_w, info.num_cores)
    zx, zy = k(x.astype(jnp.int32), y.astype(jnp.int32), emb)
    return (zx, zy, action)


# speedup vs baseline: 1.2997x; 1.0178x over previous
import functools

import jax
import jax.numpy as jnp
from jax import lax
from jax.experimental import pallas as pl
from jax.experimental.pallas import tpu as pltpu
from jax.experimental.pallas import tpu_sc as plsc


def _gather_kernel(B, D, b_per_w, num_cores):
    mesh = plsc.VectorSubcoreMesh(core_axis_name="c", subcore_axis_name="s")

    @functools.partial(
        pl.kernel,
        mesh=mesh,
        out_type=(
            jax.ShapeDtypeStruct((B, D), jnp.float32),
            jax.ShapeDtypeStruct((B, D), jnp.float32),
        ),
        scratch_types=[
            pltpu.VMEM((b_per_w,), jnp.int32),
            pltpu.VMEM((b_per_w,), jnp.int32),
            pltpu.VMEM((b_per_w, D), jnp.float32),
            pltpu.SemaphoreType.DMA,
            pltpu.SemaphoreType.DMA,
        ],
    )
    def k(x_hbm, y_hbm, emb_hbm, zx_hbm, zy_hbm, xidx_v, yidx_v, rows_v, sem, isem):
        wid = lax.axis_index("s") * num_cores + lax.axis_index("c")
        base = wid * b_per_w
        cx = pltpu.async_copy(x_hbm.at[pl.ds(base, b_per_w)], xidx_v, sem)
        cy = pltpu.async_copy(y_hbm.at[pl.ds(base, b_per_w)], yidx_v, isem)
        cx.wait()
        pltpu.async_copy(emb_hbm.at[xidx_v], rows_v, sem).wait()
        pltpu.sync_copy(rows_v, zx_hbm.at[pl.ds(base, b_per_w)])
        cy.wait()
        pltpu.async_copy(emb_hbm.at[yidx_v], rows_v, sem).wait()
        pltpu.sync_copy(rows_v, zy_hbm.at[pl.ds(base, b_per_w)])

    return k


def kernel(x, y, action, emb):
    (B,) = x.shape
    V, D = emb.shape
    info = plsc.get_sparse_core_info()
    nw = info.num_cores * info.num_subcores
    b_per_w = B // nw
    k = _gather_kernel(B, D, b_per_w, info.num_cores)
    zx, zy = k(x.astype(jnp.int32), y.astype(jnp.int32), emb)
    return (zx, zy, action)


# speedup vs baseline: 1.3315x; 1.0244x over previous
import functools

import jax
import jax.numpy as jnp
from jax import lax
from jax.experimental import pallas as pl
from jax.experimental.pallas import tpu as pltpu
from jax.experimental.pallas import tpu_sc as plsc

_SPLIT = 496


def _gather_kernel(B, D, b_per_w, num_cores):
    n0 = _SPLIT
    n1 = b_per_w - n0
    mesh = plsc.VectorSubcoreMesh(core_axis_name="c", subcore_axis_name="s")

    @functools.partial(
        pl.kernel,
        mesh=mesh,
        out_type=(
            jax.ShapeDtypeStruct((B, D), jnp.float32),
            jax.ShapeDtypeStruct((B, D), jnp.float32),
        ),
        scratch_types=[
            pltpu.VMEM((b_per_w,), jnp.int32),
            pltpu.VMEM((b_per_w,), jnp.int32),
            pltpu.VMEM((b_per_w, D), jnp.float32),
            pltpu.VMEM((n0, D), jnp.float32),
            pltpu.SemaphoreType.DMA,
            pltpu.SemaphoreType.DMA,
        ],
    )
    def k(x_hbm, y_hbm, emb_hbm, zx_hbm, zy_hbm, xidx_v, yidx_v, bufx, bufy, sa, sb):
        wid = lax.axis_index("s") * num_cores + lax.axis_index("c")
        base = wid * b_per_w
        cx = pltpu.async_copy(x_hbm.at[pl.ds(base, b_per_w)], xidx_v, sa)
        cy = pltpu.async_copy(y_hbm.at[pl.ds(base, b_per_w)], yidx_v, sb)
        cx.wait()
        gx = pltpu.async_copy(emb_hbm.at[xidx_v], bufx, sa)
        cy.wait()
        gy0 = pltpu.async_copy(emb_hbm.at[yidx_v.at[pl.ds(0, n0)]], bufy, sb)
        gx.wait()
        sx = pltpu.async_copy(bufx, zx_hbm.at[pl.ds(base, b_per_w)], sa)
        sx.wait()
        gy1 = pltpu.async_copy(
            emb_hbm.at[yidx_v.at[pl.ds(n0, n1)]], bufx.at[pl.ds(0, n1)], sa
        )
        gy0.wait()
        sy0 = pltpu.async_copy(bufy, zy_hbm.at[pl.ds(base, n0)], sb)
        gy1.wait()
        sy1 = pltpu.async_copy(
            bufx.at[pl.ds(0, n1)], zy_hbm.at[pl.ds(base + n0, n1)], sa
        )
        sy0.wait()
        sy1.wait()

    return k


def kernel(x, y, action, emb):
    (B,) = x.shape
    V, D = emb.shape
    info = plsc.get_sparse_core_info()
    nw = info.num_cores * info.num_subcores
    b_per_w = B // nw
    k = _gather_kernel(B, D, b_per_w, info.num_cores)
    zx, zy = k(x.astype(jnp.int32), y.astype(jnp.int32), emb)
    return (zx, zy, action)
